# Initial kernel scaffold; baseline (speedup 1.0000x reference)
import jax
import jax.numpy as jnp
from jax.experimental import pallas as pl


def kernel(hx, edge_index, edgefeats, W1, b1, W2, b2, W_ih, W_hh, b_ih, b_hh):
    raise NotImplementedError("write your pallas kernel here")


# speedup vs baseline: 2.4447x; 2.4447x over previous
import functools

import jax
import jax.numpy as jnp
from jax import lax
from jax.experimental import pallas as pl
from jax.experimental.pallas import tpu as pltpu
from jax.experimental.pallas import tpu_sc as plsc

N = 10000
E = 320000
F = 128
DE = 16
HID = 128
R = 10

NC = 2
NS = 16
CH = 128
NCH_PER_TEC = 79
E_PAD = NC * NS * CH * NCH_PER_TEC
N_PAD = 10240

_mesh = plsc.VectorSubcoreMesh(core_axis_name="c", subcore_axis_name="s")


@functools.partial(
    pl.kernel,
    out_type=jax.ShapeDtypeStruct((NC, N_PAD, F), jnp.float32),
    mesh=_mesh,
    scratch_types=[
        pltpu.VMEM((CH,), jnp.int32),
        pltpu.VMEM((CH,), jnp.int32),
        pltpu.VMEM((CH, F), jnp.float32),
        pltpu.VMEM((CH, F), jnp.float32),
        pltpu.VMEM_SHARED((N_PAD, F), jnp.float32),
        pltpu.SemaphoreType.DMA,
    ],
)
def _msg_pass(h_hbm, w_hbm, src_hbm, dst_hbm, out_hbm,
              sidx, didx, wbuf, rbuf, agg_sh, sem):
    c = lax.axis_index("c")
    s = lax.axis_index("s")
    tec = c * NS + s

    def _zrow(r, _):
        for k in range(F // 16):
            wbuf[r, pl.ds(k * 16, 16)] = jnp.zeros((16,), jnp.float32)
        return 0
    lax.fori_loop(0, CH, _zrow, 0)
    rows_per_tec = N_PAD // NS
    base_row = s * rows_per_tec
    for b in range(rows_per_tec // CH):
        pltpu.sync_copy(wbuf, agg_sh.at[pl.ds(base_row + b * CH, CH)])
    plsc.subcore_barrier()

    chunk0 = tec * NCH_PER_TEC

    def _chunk(i, _):
        ci = chunk0 + i
        e0 = ci * CH
        pltpu.sync_copy(src_hbm.at[ci], sidx)
        pltpu.sync_copy(dst_hbm.at[ci], didx)
        pltpu.sync_copy(w_hbm.at[pl.ds(e0, CH)], wbuf)
        pltpu.async_copy(h_hbm.at[sidx], rbuf, sem).wait()

        def _mrow(r, _):
            for k in range(F // 16):
                sl = pl.ds(k * 16, 16)
                rbuf[r, sl] = rbuf[r, sl] * wbuf[r, sl]
            return 0
        lax.fori_loop(0, CH, _mrow, 0)
        pltpu.sync_copy(rbuf, agg_sh.at[didx], add=True)
        return 0

    lax.fori_loop(0, NCH_PER_TEC, _chunk, 0)
    plsc.subcore_barrier()

    for b in range(rows_per_tec // CH):
        r0 = base_row + b * CH
        pltpu.sync_copy(agg_sh.at[pl.ds(r0, CH)], out_hbm.at[c, pl.ds(r0, CH)])


@functools.partial(
    pl.kernel,
    out_type=jax.ShapeDtypeStruct((NC, N_PAD, F), jnp.float32),
    mesh=_mesh,
    scratch_types=[
        pltpu.VMEM((CH,), jnp.int32),
        pltpu.VMEM((CH, F), jnp.float32),
        pltpu.VMEM((CH, F), jnp.float32),
        pltpu.VMEM_SHARED((N_PAD, F), jnp.float32),
    ],
)
def _degree(dst_hbm, out_hbm, didx, ones_b, zero_b, deg_sh):
    c = lax.axis_index("c")
    s = lax.axis_index("s")
    tec = c * NS + s

    def _fill(r, _):
        for k in range(F // 16):
            ones_b[r, pl.ds(k * 16, 16)] = jnp.ones((16,), jnp.float32)
            zero_b[r, pl.ds(k * 16, 16)] = jnp.zeros((16,), jnp.float32)
        return 0
    lax.fori_loop(0, CH, _fill, 0)
    rows_per_tec = N_PAD // NS
    base_row = s * rows_per_tec
    for b in range(rows_per_tec // CH):
        pltpu.sync_copy(zero_b, deg_sh.at[pl.ds(base_row + b * CH, CH)])
    plsc.subcore_barrier()

    chunk0 = tec * NCH_PER_TEC

    def _chunk(i, _):
        pltpu.sync_copy(dst_hbm.at[chunk0 + i], didx)
        pltpu.sync_copy(ones_b, deg_sh.at[didx], add=True)
        return 0
    lax.fori_loop(0, NCH_PER_TEC, _chunk, 0)
    plsc.subcore_barrier()

    for b in range(rows_per_tec // CH):
        r0 = base_row + b * CH
        pltpu.sync_copy(deg_sh.at[pl.ds(r0, CH)], out_hbm.at[c, pl.ds(r0, CH)])


_BE = 512


def _filter_body(ef_ref, w1_ref, b1_ref, w2_ref, b2_ref, out_ref):
    hmid = jnp.maximum(
        jnp.dot(ef_ref[...], w1_ref[...], preferred_element_type=jnp.float32)
        + b1_ref[...], 0.0)
    out_ref[...] = (
        jnp.dot(hmid, w2_ref[...], preferred_element_type=jnp.float32)
        + b2_ref[...])


_filter_call = pl.pallas_call(
    _filter_body,
    out_shape=jax.ShapeDtypeStruct((E_PAD, F), jnp.float32),
    grid=(E_PAD // _BE,),
    in_specs=[
        pl.BlockSpec((_BE, DE), lambda i: (i, 0)),
        pl.BlockSpec((DE, HID), lambda i: (0, 0)),
        pl.BlockSpec((1, HID), lambda i: (0, 0)),
        pl.BlockSpec((HID, F), lambda i: (0, 0)),
        pl.BlockSpec((1, F), lambda i: (0, 0)),
    ],
    out_specs=pl.BlockSpec((_BE, F), lambda i: (i, 0)),
)


_BN = 256


def _gru_body(agg_ref, deg_ref, h_ref, wih_ref, whh_ref, bih_ref, bhh_ref,
              out_ref):
    a = agg_ref[0] + agg_ref[1]
    dg = deg_ref[0, :, :1] + deg_ref[1, :, :1]
    x = a / jnp.maximum(dg, 1.0)
    h = h_ref[...]
    gi = jnp.dot(x, wih_ref[...], preferred_element_type=jnp.float32) + bih_ref[...]
    gh = jnp.dot(h, whh_ref[...], preferred_element_type=jnp.float32) + bhh_ref[...]
    r = jax.nn.sigmoid(gi[:, :F] + gh[:, :F])
    z = jax.nn.sigmoid(gi[:, F:2 * F] + gh[:, F:2 * F])
    n = jnp.tanh(gi[:, 2 * F:] + r * gh[:, 2 * F:])
    out_ref[...] = (1.0 - z) * n + z * h


_gru_call = pl.pallas_call(
    _gru_body,
    out_shape=jax.ShapeDtypeStruct((N_PAD, F), jnp.float32),
    grid=(N_PAD // _BN,),
    in_specs=[
        pl.BlockSpec((NC, _BN, F), lambda i: (0, i, 0)),
        pl.BlockSpec((NC, _BN, F), lambda i: (0, i, 0)),
        pl.BlockSpec((_BN, F), lambda i: (i, 0)),
        pl.BlockSpec((F, 3 * F), lambda i: (0, 0)),
        pl.BlockSpec((F, 3 * F), lambda i: (0, 0)),
        pl.BlockSpec((1, 3 * F), lambda i: (0, 0)),
        pl.BlockSpec((1, 3 * F), lambda i: (0, 0)),
    ],
    out_specs=pl.BlockSpec((_BN, F), lambda i: (i, 0)),
)


def kernel(hx, edge_index, edgefeats, W1, b1, W2, b2, W_ih, W_hh, b_ih, b_hh):
    src = edge_index[0].astype(jnp.int32)
    dst = edge_index[1].astype(jnp.int32)
    pad = E_PAD - E
    src_p = jnp.concatenate([src, jnp.zeros((pad,), jnp.int32)])
    dst_p = jnp.concatenate([dst, jnp.full((pad,), N, jnp.int32)])
    src2d = src_p.reshape(E_PAD // CH, CH)
    dst2d = dst_p.reshape(E_PAD // CH, CH)
    ef_p = jnp.pad(edgefeats, ((0, pad), (0, 0)))

    weights = _filter_call(ef_p, W1, b1.reshape(1, HID), W2, b2.reshape(1, F))
    deg = _degree(dst2d)

    h0 = jnp.pad(hx, ((0, N_PAD - N), (0, 0)))
    bih2 = b_ih.reshape(1, 3 * F)
    bhh2 = b_hh.reshape(1, 3 * F)

    def _body(_, h):
        agg = _msg_pass(h, weights, src2d, dst2d)
        return _gru_call(agg, deg, h, W_ih, W_hh, bih2, bhh2)

    h = lax.fori_loop(0, R, _body, h0)
    return h[:N]


# speedup vs baseline: 3.1871x; 1.3037x over previous
import functools

import jax
import jax.numpy as jnp
from jax import lax
from jax.experimental import pallas as pl
from jax.experimental.pallas import tpu as pltpu
from jax.experimental.pallas import tpu_sc as plsc

N = 10000
E = 320000
F = 128
DE = 16
HID = 128
R = 10

NC = 2
NS = 16
CH = 96
NCH_PER_TEC = 106
E_PAD = NC * NS * CH * NCH_PER_TEC
N_TBL = 10112
N_PAD = 10240


_mesh = plsc.VectorSubcoreMesh(core_axis_name="c", subcore_axis_name="s")


@functools.partial(
    pl.kernel,
    out_type=jax.ShapeDtypeStruct((NC, N_PAD, F), jnp.float32),
    mesh=_mesh,
    scratch_types=[
        pltpu.VMEM((2, CH), jnp.int32),
        pltpu.VMEM((2, CH), jnp.int32),
        pltpu.VMEM((2, CH, F), jnp.float32),
        pltpu.VMEM((2, CH, F), jnp.float32),
        pltpu.VMEM_SHARED((N_TBL, F), jnp.float32),
        pltpu.SemaphoreType.DMA,
        pltpu.SemaphoreType.DMA,
        pltpu.SemaphoreType.DMA,
        pltpu.SemaphoreType.DMA,
    ],
)
def _msg_pass(h_hbm, w_hbm, src_hbm, dst_hbm, out_hbm,
              sidx, didx, wbuf, rbuf, agg_sh, sem0, sem1, isem0, isem1):
    c = lax.axis_index("c")
    s = lax.axis_index("s")
    sems = (sem0, sem1)
    isems = (isem0, isem1)

    def _zrow(r, _):
        for k in range(F // 16):
            wbuf[0, r, pl.ds(k * 16, 16)] = jnp.zeros((16,), jnp.float32)
        return 0
    lax.fori_loop(0, CH, _zrow, 0)
    rows_per_tec = N_TBL // NS
    base_row = s * rows_per_tec
    nfull, rem = divmod(rows_per_tec, CH)
    for b in range(nfull):
        pltpu.sync_copy(wbuf.at[0], agg_sh.at[pl.ds(base_row + b * CH, CH)])
    if rem:
        pltpu.sync_copy(wbuf.at[0, pl.ds(0, rem)],
                        agg_sh.at[pl.ds(base_row + nfull * CH, rem)])
    plsc.subcore_barrier()

    tec = c * NS + s
    chunk0 = tec * NCH_PER_TEC

    def _start_idx(i, b):
        pltpu.async_copy(src_hbm.at[chunk0 + i], sidx.at[b], isems[b])
        pltpu.async_copy(dst_hbm.at[chunk0 + i], didx.at[b], isems[b])

    def _wait_idx(b):
        pltpu.make_async_copy(src_hbm.at[0], sidx.at[b], isems[b]).wait()
        pltpu.make_async_copy(src_hbm.at[0], didx.at[b], isems[b]).wait()

    def _start(i, b):
        e0 = (chunk0 + i) * CH
        pltpu.async_copy(w_hbm.at[pl.ds(e0, CH)], wbuf.at[b], sems[b])
        pltpu.async_copy(h_hbm.at[sidx.at[b]], rbuf.at[b], sems[b])

    def _wait(b):
        pltpu.make_async_copy(w_hbm.at[pl.ds(0, CH)], wbuf.at[b], sems[b]).wait()
        pltpu.make_async_copy(w_hbm.at[pl.ds(0, CH)], rbuf.at[b], sems[b]).wait()

    _start_idx(0, 0)
    _start_idx(1, 1)
    _wait_idx(0)
    _start(0, 0)

    def _g(g, _):
        for b in range(2):
            i = g * 2 + b

            @pl.when(i + 1 < NCH_PER_TEC)
            def _():
                _wait_idx(1 - b)
                _start(i + 1, 1 - b)

            _wait(b)

            def _mrow(r, _):
                for k in range(F // 16):
                    sl = pl.ds(k * 16, 16)
                    rbuf[b, r, sl] = rbuf[b, r, sl] * wbuf[b, r, sl]
                return 0
            lax.fori_loop(0, CH, _mrow, 0)
            pltpu.sync_copy(rbuf.at[b], agg_sh.at[didx.at[b]], add=True)

            @pl.when(i + 2 < NCH_PER_TEC)
            def _():
                _start_idx(i + 2, b)
        return 0

    lax.fori_loop(0, NCH_PER_TEC // 2, _g, 0)
    plsc.subcore_barrier()

    for b in range(nfull):
        r0 = base_row + b * CH
        pltpu.sync_copy(agg_sh.at[pl.ds(r0, CH)], out_hbm.at[c, pl.ds(r0, CH)])
    if rem:
        r0 = base_row + nfull * CH
        pltpu.sync_copy(agg_sh.at[pl.ds(r0, rem)], out_hbm.at[c, pl.ds(r0, rem)])


@functools.partial(
    pl.kernel,
    out_type=jax.ShapeDtypeStruct((NC, N_PAD, F), jnp.float32),
    mesh=_mesh,
    scratch_types=[
        pltpu.VMEM((CH,), jnp.int32),
        pltpu.VMEM((CH, F), jnp.float32),
        pltpu.VMEM((CH, F), jnp.float32),
        pltpu.VMEM_SHARED((N_TBL, F), jnp.float32),
    ],
)
def _degree(dst_hbm, out_hbm, didx, ones_b, zero_b, deg_sh):
    c = lax.axis_index("c")
    s = lax.axis_index("s")
    tec = c * NS + s

    def _fill(r, _):
        for k in range(F // 16):
            ones_b[r, pl.ds(k * 16, 16)] = jnp.ones((16,), jnp.float32)
            zero_b[r, pl.ds(k * 16, 16)] = jnp.zeros((16,), jnp.float32)
        return 0
    lax.fori_loop(0, CH, _fill, 0)
    rows_per_tec = N_TBL // NS
    base_row = s * rows_per_tec
    nfull, rem = divmod(rows_per_tec, CH)
    for b in range(nfull):
        pltpu.sync_copy(zero_b, deg_sh.at[pl.ds(base_row + b * CH, CH)])
    if rem:
        pltpu.sync_copy(zero_b.at[pl.ds(0, rem)],
                        deg_sh.at[pl.ds(base_row + nfull * CH, rem)])
    plsc.subcore_barrier()

    chunk0 = tec * NCH_PER_TEC

    def _chunk(i, _):
        pltpu.sync_copy(dst_hbm.at[chunk0 + i], didx)
        pltpu.sync_copy(ones_b, deg_sh.at[didx], add=True)
        return 0
    lax.fori_loop(0, NCH_PER_TEC, _chunk, 0)
    plsc.subcore_barrier()

    for b in range(nfull):
        r0 = base_row + b * CH
        pltpu.sync_copy(deg_sh.at[pl.ds(r0, CH)], out_hbm.at[c, pl.ds(r0, CH)])
    if rem:
        r0 = base_row + nfull * CH
        pltpu.sync_copy(deg_sh.at[pl.ds(r0, rem)], out_hbm.at[c, pl.ds(r0, rem)])


_BE = 512


def _filter_body(ef_ref, w1_ref, b1_ref, w2_ref, b2_ref, out_ref):
    hmid = jnp.maximum(
        jnp.dot(ef_ref[...], w1_ref[...], preferred_element_type=jnp.float32)
        + b1_ref[...], 0.0)
    out_ref[...] = (
        jnp.dot(hmid, w2_ref[...], preferred_element_type=jnp.float32)
        + b2_ref[...])


_filter_call = pl.pallas_call(
    _filter_body,
    out_shape=jax.ShapeDtypeStruct((E_PAD, F), jnp.float32),
    grid=(E_PAD // _BE,),
    in_specs=[
        pl.BlockSpec((_BE, DE), lambda i: (i, 0)),
        pl.BlockSpec((DE, HID), lambda i: (0, 0)),
        pl.BlockSpec((1, HID), lambda i: (0, 0)),
        pl.BlockSpec((HID, F), lambda i: (0, 0)),
        pl.BlockSpec((1, F), lambda i: (0, 0)),
    ],
    out_specs=pl.BlockSpec((_BE, F), lambda i: (i, 0)),
)


_BN = 256


def _gru_body(agg_ref, deg_ref, h_ref, wih_ref, whh_ref, bih_ref, bhh_ref,
              out_ref):
    a = agg_ref[0] + agg_ref[1]
    dg = deg_ref[0, :, :1] + deg_ref[1, :, :1]
    x = a / jnp.maximum(dg, 1.0)
    h = h_ref[...]
    gi = jnp.dot(x, wih_ref[...], preferred_element_type=jnp.float32) + bih_ref[...]
    gh = jnp.dot(h, whh_ref[...], preferred_element_type=jnp.float32) + bhh_ref[...]
    r = jax.nn.sigmoid(gi[:, :F] + gh[:, :F])
    z = jax.nn.sigmoid(gi[:, F:2 * F] + gh[:, F:2 * F])
    n = jnp.tanh(gi[:, 2 * F:] + r * gh[:, 2 * F:])
    out_ref[...] = (1.0 - z) * n + z * h


_gru_call = pl.pallas_call(
    _gru_body,
    out_shape=jax.ShapeDtypeStruct((N_PAD, F), jnp.float32),
    grid=(N_PAD // _BN,),
    in_specs=[
        pl.BlockSpec((NC, _BN, F), lambda i: (0, i, 0)),
        pl.BlockSpec((NC, _BN, F), lambda i: (0, i, 0)),
        pl.BlockSpec((_BN, F), lambda i: (i, 0)),
        pl.BlockSpec((F, 3 * F), lambda i: (0, 0)),
        pl.BlockSpec((F, 3 * F), lambda i: (0, 0)),
        pl.BlockSpec((1, 3 * F), lambda i: (0, 0)),
        pl.BlockSpec((1, 3 * F), lambda i: (0, 0)),
    ],
    out_specs=pl.BlockSpec((_BN, F), lambda i: (i, 0)),
)


def kernel(hx, edge_index, edgefeats, W1, b1, W2, b2, W_ih, W_hh, b_ih, b_hh):
    src = edge_index[0].astype(jnp.int32)
    dst = edge_index[1].astype(jnp.int32)
    pad = E_PAD - E
    src_p = jnp.concatenate([src, jnp.zeros((pad,), jnp.int32)])
    dst_p = jnp.concatenate([dst, jnp.full((pad,), N, jnp.int32)])
    src2d = src_p.reshape(E_PAD // CH, CH)
    dst2d = dst_p.reshape(E_PAD // CH, CH)
    ef_p = jnp.pad(edgefeats, ((0, pad), (0, 0)))

    weights = _filter_call(ef_p, W1, b1.reshape(1, HID), W2, b2.reshape(1, F))
    deg = _degree(dst2d)

    h0 = jnp.pad(hx, ((0, N_PAD - N), (0, 0)))
    bih2 = b_ih.reshape(1, 3 * F)
    bhh2 = b_hh.reshape(1, 3 * F)

    def _body(_, h):
        agg = _msg_pass(h, weights, src2d, dst2d)
        return _gru_call(agg, deg, h, W_ih, W_hh, bih2, bhh2)

    h = lax.fori_loop(0, R, _body, h0)
    return h[:N]
